# baseline (speedup 1.0000x reference)
import functools

import jax
import jax.numpy as jnp
from jax import lax
from jax.experimental import pallas as pl
from jax.experimental.pallas import tpu as pltpu
from jax.experimental.pallas import tpu_sc as plsc

_NUM_CORES = 2
_NUM_SUBCORES = 16
_LANES = 16
_NW = _NUM_CORES * _NUM_SUBCORES

_VOC_PAD = 256


def _histogram_sc(pos_ids, dep_ids, dep_offset):
    B, S = pos_ids.shape
    rows_per_w = B // _NW
    CH = _LANES
    n_chunks = rows_per_w // CH

    mesh = plsc.VectorSubcoreMesh(core_axis_name="c", subcore_axis_name="s")

    @functools.partial(
        pl.kernel,
        out_type=jax.ShapeDtypeStruct((B, _VOC_PAD), jnp.float32),
        mesh=mesh,
        compiler_params=pltpu.CompilerParams(use_tc_tiling_on_sc=False,
                                             needs_layout_passes=False),
        scratch_types=[
            pltpu.VMEM((CH, S), jnp.int32),
            pltpu.VMEM((CH, S), jnp.int32),
            pltpu.VMEM((CH, _VOC_PAD), jnp.float32),
        ],
    )
    def hist(pos_hbm, dep_hbm, out_hbm, pos_v, dep_v, cnt_v):
        wid = lax.axis_index("s") * _NUM_CORES + lax.axis_index("c")
        iota = lax.iota(jnp.int32, _LANES)
        ones = jnp.ones((_LANES,), jnp.float32)
        zeros = jnp.zeros((_LANES,), jnp.float32)

        UNROLL = 8
        assert S % UNROLL == 0

        def chunk_body(c, carry):
            base = wid * rows_per_w + c * CH
            pltpu.sync_copy(pos_hbm.at[pl.ds(base, CH), :], pos_v)
            pltpu.sync_copy(dep_hbm.at[pl.ds(base, CH), :], dep_v)

            for i in range(CH):
                for j in range(_VOC_PAD // _LANES):
                    cnt_v[i, pl.ds(j * _LANES, _LANES)] = zeros

            def s_body(k, carry2):
                sbase = jnp.broadcast_to(k * UNROLL, (_LANES,)).astype(
                    jnp.int32)
                for u in range(UNROLL):
                    scol = sbase + u
                    p = plsc.load_gather(pos_v, [iota, scol])
                    plsc.addupdate_scatter(cnt_v, [iota, p], ones)
                    d = plsc.load_gather(dep_v, [iota, scol]) + dep_offset
                    plsc.addupdate_scatter(cnt_v, [iota, d], ones)
                return carry2

            lax.fori_loop(0, S // UNROLL, s_body, 0)
            pltpu.sync_copy(cnt_v, out_hbm.at[pl.ds(base, CH), :])
            return carry

        lax.fori_loop(0, n_chunks, chunk_body, 0)

    return hist(pos_ids, dep_ids)


def _finish_tc(counts, T_cat, W, b2, inv_s):
    B = counts.shape[0]
    OD, TWO_ED = W.shape
    BT = 1024

    def body(cnt_ref, t_ref, w_ref, b_ref, o_ref):
        comb = jnp.dot(cnt_ref[...], t_ref[...],
                       preferred_element_type=jnp.float32) * inv_s
        out = lax.dot_general(comb, w_ref[...],
                              dimension_numbers=(((1,), (1,)), ((), ())),
                              preferred_element_type=jnp.float32)
        o_ref[...] = jnp.maximum(out + b_ref[...], 0.0)

    return pl.pallas_call(
        body,
        grid=(B // BT,),
        in_specs=[
            pl.BlockSpec((BT, _VOC_PAD), lambda i: (i, 0)),
            pl.BlockSpec((_VOC_PAD, TWO_ED), lambda i: (0, 0)),
            pl.BlockSpec((OD, TWO_ED), lambda i: (0, 0)),
            pl.BlockSpec((1, OD), lambda i: (0, 0)),
        ],
        out_specs=pl.BlockSpec((BT, OD), lambda i: (i, 0)),
        out_shape=jax.ShapeDtypeStruct((B, OD), jnp.float32),
    )(counts, T_cat, W, b2)


def kernel(pos_ids, dep_ids, pos_table, dep_table, W, b):
    S = pos_ids.shape[1]
    NP, ED = pos_table.shape
    ND = dep_table.shape[1 - 1]

    counts = _histogram_sc(pos_ids.astype(jnp.int32),
                           dep_ids.astype(jnp.int32), NP)

    T_cat = jnp.zeros((_VOC_PAD, 2 * ED), jnp.float32)
    T_cat = T_cat.at[0:NP, 0:ED].set(pos_table)
    T_cat = T_cat.at[NP:NP + ND, ED:2 * ED].set(dep_table)

    return _finish_tc(counts, T_cat, W.astype(jnp.float32),
                      b.astype(jnp.float32).reshape(1, -1), 1.0 / S)


# speedup vs baseline: 1.1293x; 1.1293x over previous
import functools

import jax
import jax.numpy as jnp
from jax import lax
from jax.experimental import pallas as pl
from jax.experimental.pallas import tpu as pltpu
from jax.experimental.pallas import tpu_sc as plsc

_NUM_CORES = 2
_NUM_SUBCORES = 16
_LANES = 16
_NW = _NUM_CORES * _NUM_SUBCORES

_VOC_PAD = 256


def _histogram_sc(pos_ids, dep_ids, dep_offset):
    B, S = pos_ids.shape
    rows_per_w = B // _NW
    SUPER = 128
    n_super = rows_per_w // SUPER
    n_sub = SUPER // _LANES

    mesh = plsc.VectorSubcoreMesh(core_axis_name="c", subcore_axis_name="s")

    @functools.partial(
        pl.kernel,
        out_type=jax.ShapeDtypeStruct((B, _VOC_PAD), jnp.float32),
        mesh=mesh,
        compiler_params=pltpu.CompilerParams(use_tc_tiling_on_sc=False,
                                             needs_layout_passes=False),
        scratch_types=[
            pltpu.VMEM((SUPER, S), jnp.int32),
            pltpu.VMEM((SUPER, S), jnp.int32),
            pltpu.VMEM((SUPER, _VOC_PAD), jnp.float32),
        ],
    )
    def hist(pos_hbm, dep_hbm, out_hbm, pos_v, dep_v, cnt_v):
        wid = lax.axis_index("s") * _NUM_CORES + lax.axis_index("c")
        iota = lax.iota(jnp.int32, _LANES)
        ones = jnp.ones((_LANES,), jnp.float32)
        zeros = jnp.zeros((_LANES,), jnp.float32)

        UNROLL = 8
        assert S % UNROLL == 0

        def super_body(c, carry):
            base = wid * rows_per_w + c * SUPER
            pltpu.sync_copy(pos_hbm.at[pl.ds(base, SUPER), :], pos_v)
            pltpu.sync_copy(dep_hbm.at[pl.ds(base, SUPER), :], dep_v)

            def zero_body(j, carry2):
                for i in range(SUPER):
                    cnt_v[i, pl.ds(j * _LANES, _LANES)] = zeros
                return carry2

            lax.fori_loop(0, _VOC_PAD // _LANES, zero_body, 0)

            for sub in range(n_sub):
                rowv = iota + sub * _LANES

                def s_body(k, carry2):
                    sbase = jnp.broadcast_to(k * UNROLL, (_LANES,)).astype(
                        jnp.int32)
                    for u in range(UNROLL):
                        scol = sbase + u
                        p = plsc.load_gather(pos_v, [rowv, scol])
                        plsc.addupdate_scatter(cnt_v, [rowv, p], ones)
                        d = plsc.load_gather(dep_v, [rowv, scol]) + dep_offset
                        plsc.addupdate_scatter(cnt_v, [rowv, d], ones)
                    return carry2

                lax.fori_loop(0, S // UNROLL, s_body, 0)

            pltpu.sync_copy(cnt_v, out_hbm.at[pl.ds(base, SUPER), :])
            return carry

        lax.fori_loop(0, n_super, super_body, 0)

    return hist(pos_ids, dep_ids)


def _finish_tc(counts, T_cat, W, b2, inv_s):
    B = counts.shape[0]
    OD, TWO_ED = W.shape
    BT = 1024

    def body(cnt_ref, t_ref, w_ref, b_ref, o_ref):
        comb = jnp.dot(cnt_ref[...], t_ref[...],
                       preferred_element_type=jnp.float32) * inv_s
        out = lax.dot_general(comb, w_ref[...],
                              dimension_numbers=(((1,), (1,)), ((), ())),
                              preferred_element_type=jnp.float32)
        o_ref[...] = jnp.maximum(out + b_ref[...], 0.0)

    return pl.pallas_call(
        body,
        grid=(B // BT,),
        in_specs=[
            pl.BlockSpec((BT, _VOC_PAD), lambda i: (i, 0)),
            pl.BlockSpec((_VOC_PAD, TWO_ED), lambda i: (0, 0)),
            pl.BlockSpec((OD, TWO_ED), lambda i: (0, 0)),
            pl.BlockSpec((1, OD), lambda i: (0, 0)),
        ],
        out_specs=pl.BlockSpec((BT, OD), lambda i: (i, 0)),
        out_shape=jax.ShapeDtypeStruct((B, OD), jnp.float32),
    )(counts, T_cat, W, b2)


def kernel(pos_ids, dep_ids, pos_table, dep_table, W, b):
    S = pos_ids.shape[1]
    NP, ED = pos_table.shape
    ND = dep_table.shape[1 - 1]

    counts = _histogram_sc(pos_ids.astype(jnp.int32),
                           dep_ids.astype(jnp.int32), NP)

    T_cat = jnp.zeros((_VOC_PAD, 2 * ED), jnp.float32)
    T_cat = T_cat.at[0:NP, 0:ED].set(pos_table)
    T_cat = T_cat.at[NP:NP + ND, ED:2 * ED].set(dep_table)

    return _finish_tc(counts, T_cat, W.astype(jnp.float32),
                      b.astype(jnp.float32).reshape(1, -1), 1.0 / S)


# speedup vs baseline: 1.7364x; 1.5376x over previous
import functools

import jax
import jax.numpy as jnp
from jax import lax
from jax.experimental import pallas as pl
from jax.experimental.pallas import tpu as pltpu
from jax.experimental.pallas import tpu_sc as plsc

_NUM_CORES = 2
_NUM_SUBCORES = 16
_LANES = 16
_NW = _NUM_CORES * _NUM_SUBCORES

_VOC_PAD = 256


def _histogram_sc(pos_ids, dep_ids, dep_offset):
    B, S = pos_ids.shape
    rows_per_w = B // _NW
    SUPER = 128
    n_super = rows_per_w // SUPER
    n_sub = SUPER // _LANES

    mesh = plsc.VectorSubcoreMesh(core_axis_name="c", subcore_axis_name="s")

    @functools.partial(
        pl.kernel,
        out_type=jax.ShapeDtypeStruct((B, _VOC_PAD), jnp.float32),
        mesh=mesh,
        compiler_params=pltpu.CompilerParams(use_tc_tiling_on_sc=False,
                                             needs_layout_passes=False),
        scratch_types=[
            pltpu.VMEM((SUPER, S), jnp.int32),
            pltpu.VMEM((SUPER, S), jnp.int32),
            pltpu.VMEM((SUPER, _VOC_PAD), jnp.float32),
        ],
    )
    def hist(pos_hbm, dep_hbm, out_hbm, pos_v, dep_v, cnt_v):
        wid = lax.axis_index("s") * _NUM_CORES + lax.axis_index("c")
        iota = lax.iota(jnp.int32, _LANES)
        ones = jnp.ones((_LANES,), jnp.float32)
        zeros = jnp.zeros((_LANES,), jnp.float32)

        UNROLL = 8
        assert S % UNROLL == 0

        def super_body(c, carry):
            base = wid * rows_per_w + c * SUPER
            pltpu.sync_copy(pos_hbm.at[pl.ds(base, SUPER), :], pos_v)
            pltpu.sync_copy(dep_hbm.at[pl.ds(base, SUPER), :], dep_v)

            def zero_body(j, carry2):
                for i in range(SUPER):
                    cnt_v[i, pl.ds(j * _LANES, _LANES)] = zeros
                return carry2

            lax.fori_loop(0, _VOC_PAD // _LANES, zero_body, 0)

            for sub in range(n_sub):
                rowv = iota + sub * _LANES

                @plsc.parallel_loop(0, S, step=1, unroll=UNROLL)
                def s_body(s):
                    scol = jnp.broadcast_to(s, (_LANES,)).astype(jnp.int32)
                    p = plsc.load_gather(pos_v, [rowv, scol])
                    plsc.addupdate_scatter(cnt_v, [rowv, p], ones)
                    d = plsc.load_gather(dep_v, [rowv, scol]) + dep_offset
                    plsc.addupdate_scatter(cnt_v, [rowv, d], ones)

            pltpu.sync_copy(cnt_v, out_hbm.at[pl.ds(base, SUPER), :])
            return carry

        lax.fori_loop(0, n_super, super_body, 0)

    return hist(pos_ids, dep_ids)


def _finish_tc(counts, T_cat, W, b2, inv_s):
    B = counts.shape[0]
    OD, TWO_ED = W.shape
    BT = 1024

    def body(cnt_ref, t_ref, w_ref, b_ref, o_ref):
        comb = jnp.dot(cnt_ref[...], t_ref[...],
                       preferred_element_type=jnp.float32) * inv_s
        out = lax.dot_general(comb, w_ref[...],
                              dimension_numbers=(((1,), (1,)), ((), ())),
                              preferred_element_type=jnp.float32)
        o_ref[...] = jnp.maximum(out + b_ref[...], 0.0)

    return pl.pallas_call(
        body,
        grid=(B // BT,),
        in_specs=[
            pl.BlockSpec((BT, _VOC_PAD), lambda i: (i, 0)),
            pl.BlockSpec((_VOC_PAD, TWO_ED), lambda i: (0, 0)),
            pl.BlockSpec((OD, TWO_ED), lambda i: (0, 0)),
            pl.BlockSpec((1, OD), lambda i: (0, 0)),
        ],
        out_specs=pl.BlockSpec((BT, OD), lambda i: (i, 0)),
        out_shape=jax.ShapeDtypeStruct((B, OD), jnp.float32),
    )(counts, T_cat, W, b2)


def kernel(pos_ids, dep_ids, pos_table, dep_table, W, b):
    S = pos_ids.shape[1]
    NP, ED = pos_table.shape
    ND = dep_table.shape[1 - 1]

    counts = _histogram_sc(pos_ids.astype(jnp.int32),
                           dep_ids.astype(jnp.int32), NP)

    T_cat = jnp.zeros((_VOC_PAD, 2 * ED), jnp.float32)
    T_cat = T_cat.at[0:NP, 0:ED].set(pos_table)
    T_cat = T_cat.at[NP:NP + ND, ED:2 * ED].set(dep_table)

    return _finish_tc(counts, T_cat, W.astype(jnp.float32),
                      b.astype(jnp.float32).reshape(1, -1), 1.0 / S)
